# baseline (speedup 1.0000x reference)
import functools

import jax
import jax.numpy as jnp
from jax import lax
from jax.experimental import pallas as pl
from jax.experimental.pallas import tpu as pltpu
from jax.experimental.pallas import tpu_sc as plsc

N_NODES = 10000
N_EDGES = 160000
D_IN = 512
D_OUT = 256

NC = 2
NS = 16
CHUNK = 128
CHUNKS_PER_TILE = 80
E_PAD = NS * CHUNKS_PER_TILE * CHUNK
IDX_BITS = 14
DH = D_OUT // NC
STRIPE = 632
ACC_ROWS = NS * STRIPE
DUMMY_ROW = N_NODES
ROW_BLK = 400


def _matmul_body(x_ref, w_ref, o_ref):
    o_ref[0, :, :] = lax.dot_general(
        x_ref[...], w_ref[...], (((1,), (0,)), ((), ())),
        precision=lax.Precision.HIGHEST,
        preferred_element_type=jnp.float32)


def _support_split(x, W):
    grid = (NC, N_NODES // ROW_BLK)
    return pl.pallas_call(
        _matmul_body,
        grid=grid,
        in_specs=[
            pl.BlockSpec((ROW_BLK, D_IN), lambda c, i: (i, 0)),
            pl.BlockSpec((D_IN, DH), lambda c, i: (0, c)),
        ],
        out_specs=pl.BlockSpec((1, ROW_BLK, DH), lambda c, i: (c, i, 0)),
        out_shape=jax.ShapeDtypeStruct((NC, N_NODES, DH), jnp.float32),
    )(x, W)


@functools.partial(
    pl.kernel,
    mesh=plsc.VectorSubcoreMesh(core_axis_name="c", subcore_axis_name="s"),
    out_type=jax.ShapeDtypeStruct((NC, N_NODES, DH), jnp.float32),
    scratch_types=[
        pltpu.VMEM((CHUNKS_PER_TILE + 1, CHUNK), jnp.int32),
        pltpu.VMEM((CHUNK, DH), jnp.float32),
        pltpu.VMEM((CHUNK, DH), jnp.float32),
        pltpu.VMEM((CHUNK,), jnp.int32),
        pltpu.VMEM((CHUNK,), jnp.int32),
        pltpu.VMEM((CHUNK,), jnp.int32),
        pltpu.VMEM((CHUNK,), jnp.int32),
        pltpu.VMEM_SHARED((ACC_ROWS, DH), jnp.float32),
        pltpu.SemaphoreType.DMA,
        pltpu.SemaphoreType.DMA,
    ],
)
def _sc_aggregate(support_hbm, idx_hbm, zeros_hbm, out_hbm,
                  idx_v, msg0, msg1, src0, src1, dst0, dst1, acc_sh,
                  sem0, sem1):
    c = lax.axis_index("c")
    s = lax.axis_index("s")
    tbl = support_hbm.at[c]

    def unpack(row, src_r, dst_r):
        packed_row = idx_v.at[row]
        for k in range(CHUNK // 16):
            sl = pl.ds(k * 16, 16)
            p = packed_row[sl]
            src_r[sl] = lax.shift_right_logical(p, IDX_BITS)
            dst_r[sl] = lax.bitwise_and(p, (1 << IDX_BITS) - 1)

    pltpu.sync_copy(zeros_hbm.at[pl.ds(s * STRIPE, STRIPE)],
                    acc_sh.at[pl.ds(s * STRIPE, STRIPE)])
    pltpu.sync_copy(idx_hbm.at[s], idx_v)
    plsc.subcore_barrier()

    unpack(0, src0, dst0)
    pltpu.async_copy(tbl.at[src0], msg0, sem0)

    @pl.loop(0, CHUNKS_PER_TILE, step=2)
    def _(j):
        unpack(j + 1, src1, dst1)
        pltpu.async_copy(tbl.at[src1], msg1, sem1)
        pltpu.make_async_copy(tbl.at[src0], msg0, sem0).wait()
        pltpu.sync_copy(msg0, acc_sh.at[dst0], add=True)
        unpack(j + 2, src0, dst0)
        pltpu.async_copy(tbl.at[src0], msg0, sem0)
        pltpu.make_async_copy(tbl.at[src1], msg1, sem1).wait()
        pltpu.sync_copy(msg1, acc_sh.at[dst1], add=True)

    pltpu.make_async_copy(tbl.at[src0], msg0, sem0).wait()

    plsc.subcore_barrier()

    @pl.when(s < NS - 1)
    def _():
        pltpu.sync_copy(acc_sh.at[pl.ds(s * STRIPE, STRIPE)],
                        out_hbm.at[c].at[pl.ds(s * STRIPE, STRIPE)])

    @pl.when(s == NS - 1)
    def _():
        last = N_NODES - (NS - 1) * STRIPE
        pltpu.sync_copy(acc_sh.at[pl.ds((NS - 1) * STRIPE, last)],
                        out_hbm.at[c].at[pl.ds((NS - 1) * STRIPE, last)])


def kernel(adj, x, W):
    support = _support_split(x, W)

    pad = E_PAD - N_EDGES
    packed = jnp.concatenate(
        [adj[0] * (1 << IDX_BITS) + adj[1],
         jnp.full((pad,), DUMMY_ROW, jnp.int32)])
    idx3 = jnp.concatenate(
        [packed.reshape(NS, CHUNKS_PER_TILE, CHUNK),
         jnp.full((NS, 1, CHUNK), DUMMY_ROW, jnp.int32)], axis=1)
    zeros = jnp.zeros((ACC_ROWS, DH), jnp.float32)

    out2 = _sc_aggregate(support, idx3, zeros)
    return jnp.concatenate([out2[0], out2[1]], axis=1)


# speedup vs baseline: 1.0868x; 1.0868x over previous
import functools

import jax
import jax.numpy as jnp
from jax import lax
from jax.experimental import pallas as pl
from jax.experimental.pallas import tpu as pltpu
from jax.experimental.pallas import tpu_sc as plsc

N_NODES = 10000
N_EDGES = 160000
D_IN = 512
D_OUT = 256

NC = 2
NS = 16
CHUNK = 128
CHUNKS_PER_TILE = 80
E_PAD = NS * CHUNKS_PER_TILE * CHUNK
IDX_BITS = 14
DH = D_OUT // NC
STRIPE = 632
ACC_ROWS = NS * STRIPE
DUMMY_ROW = N_NODES
ROW_BLK = 400


def _matmul_body(x_ref, w_ref, o_ref):
    o_ref[0, :, :] = lax.dot_general(
        x_ref[...], w_ref[...], (((1,), (0,)), ((), ())),
        precision=lax.Precision.HIGHEST,
        preferred_element_type=jnp.float32)


def _support_split(x, W):
    grid = (NC, N_NODES // ROW_BLK)
    return pl.pallas_call(
        _matmul_body,
        grid=grid,
        in_specs=[
            pl.BlockSpec((ROW_BLK, D_IN), lambda c, i: (i, 0)),
            pl.BlockSpec((D_IN, DH), lambda c, i: (0, c)),
        ],
        out_specs=pl.BlockSpec((1, ROW_BLK, DH), lambda c, i: (c, i, 0)),
        out_shape=jax.ShapeDtypeStruct((NC, N_NODES, DH), jnp.float32),
    )(x, W)


@functools.partial(
    pl.kernel,
    mesh=plsc.VectorSubcoreMesh(core_axis_name="c", subcore_axis_name="s"),
    out_type=jax.ShapeDtypeStruct((NC, N_NODES, DH), jnp.float32),
    scratch_types=[
        pltpu.VMEM((CHUNKS_PER_TILE + 1, CHUNK), jnp.int32),
        pltpu.VMEM((CHUNK, DH), jnp.float32),
        pltpu.VMEM((CHUNK, DH), jnp.float32),
        pltpu.VMEM((CHUNK,), jnp.int32),
        pltpu.VMEM((CHUNK,), jnp.int32),
        pltpu.VMEM((CHUNK,), jnp.int32),
        pltpu.VMEM((CHUNK,), jnp.int32),
        pltpu.VMEM_SHARED((ACC_ROWS, DH), jnp.float32),
        pltpu.SemaphoreType.DMA,
        pltpu.SemaphoreType.DMA,
    ],
)
def _sc_aggregate(support_hbm, idx_hbm, zeros_hbm, out_hbm,
                  idx_v, msg0, msg1, src0, src1, dst0, dst1, acc_sh,
                  sem0, sem1):
    c = lax.axis_index("c")
    s = lax.axis_index("s")
    tbl = support_hbm.at[c]

    def unpack(row, src_r, dst_r):
        packed_row = idx_v.at[row]
        for k in range(CHUNK // 16):
            sl = pl.ds(k * 16, 16)
            p = packed_row[sl]
            src_r[sl] = lax.shift_right_logical(p, IDX_BITS)
            dst_r[sl] = lax.bitwise_and(p, (1 << IDX_BITS) - 1)

    pltpu.sync_copy(zeros_hbm.at[pl.ds(s * STRIPE, STRIPE)],
                    acc_sh.at[pl.ds(s * STRIPE, STRIPE)])
    pltpu.sync_copy(idx_hbm.at[s], idx_v)
    plsc.subcore_barrier()

    @pl.loop(0, CHUNKS_PER_TILE, step=2)
    def _(j):
        unpack(j, src0, dst0)
        g0 = pltpu.async_copy(tbl.at[src0], msg0, sem0)
        unpack(j + 1, src1, dst1)
        g1 = pltpu.async_copy(tbl.at[src1], msg1, sem1)
        g0.wait()
        pltpu.sync_copy(msg0, acc_sh.at[dst0], add=True)
        g1.wait()
        pltpu.sync_copy(msg1, acc_sh.at[dst1], add=True)

    plsc.subcore_barrier()

    @pl.when(s < NS - 1)
    def _():
        pltpu.sync_copy(acc_sh.at[pl.ds(s * STRIPE, STRIPE)],
                        out_hbm.at[c].at[pl.ds(s * STRIPE, STRIPE)])

    @pl.when(s == NS - 1)
    def _():
        last = N_NODES - (NS - 1) * STRIPE
        pltpu.sync_copy(acc_sh.at[pl.ds((NS - 1) * STRIPE, last)],
                        out_hbm.at[c].at[pl.ds((NS - 1) * STRIPE, last)])


def kernel(adj, x, W):
    support = _support_split(x, W)

    pad = E_PAD - N_EDGES
    packed = jnp.concatenate(
        [adj[0] * (1 << IDX_BITS) + adj[1],
         jnp.full((pad,), DUMMY_ROW, jnp.int32)])
    idx3 = jnp.concatenate(
        [packed.reshape(NS, CHUNKS_PER_TILE, CHUNK),
         jnp.full((NS, 1, CHUNK), DUMMY_ROW, jnp.int32)], axis=1)
    zeros = jnp.zeros((ACC_ROWS, DH), jnp.float32)

    out2 = _sc_aggregate(support, idx3, zeros)
    return jnp.concatenate([out2[0], out2[1]], axis=1)
